# baseline (speedup 1.0000x reference)
import functools

import jax
import jax.numpy as jnp
from jax import lax
from jax.experimental import pallas as pl
from jax.experimental.pallas import tpu as pltpu
from jax.experimental.pallas import tpu_sc as plsc

VOCAB = 100000
EMB = 32
RS0 = 7
RADIUS = RS0 // 2
BATCH = 1024
SEQ = 200
OUT_C = 96

NW = 32
ROWS_PER_W = BATCH // NW
CHUNK = 40
NCHUNK = SEQ // CHUNK
NCH_W = ROWS_PER_W * NCHUNK
EBUF_T = SEQ + 2 * RADIUS + 2


def _sc_body(seq_hbm, emb_hbm, u_hbm, out_hbm, idx_all, ebuf, gbuf, obuf,
             gsem0, gsem1, esem0, esem1, osem0, osem1):
    c_id = lax.axis_index("c")
    s_id = lax.axis_index("s")
    wid = s_id * 2 + c_id
    base_row = wid * ROWS_PER_W
    gsem = (gsem0, gsem1)
    osem = (osem0, osem1)
    esem = (esem0, esem1)

    zero = jnp.zeros((16,), jnp.float32)
    for b in range(4):
        for i in range(RADIUS * EMB // 16):
            ebuf[b, pl.ds(i * 16, 16)] = zero
            ebuf[b, pl.ds((RADIUS + SEQ) * EMB + i * 16, 16)] = zero

    def gather_desc(ci, par):
        return pltpu.make_async_copy(
            u_hbm.at[idx_all.at[ci // NCHUNK, pl.ds((ci % NCHUNK) * CHUNK, CHUNK)]],
            gbuf.at[par], gsem[par])

    def emb_desc(row, b, par):
        return pltpu.make_async_copy(
            emb_hbm.at[pl.ds(row * (SEQ * EMB), SEQ * EMB)],
            ebuf.at[b, pl.ds(RADIUS * EMB, SEQ * EMB)], esem[par])

    out0 = base_row * SEQ * OUT_C

    def out_desc(ci, par):
        return pltpu.make_async_copy(
            obuf.at[par],
            out_hbm.at[pl.ds(out0 + ci * (CHUNK * OUT_C), CHUNK * OUT_C)],
            osem[par])

    def compute_chunk(b, l0, g, o):
        w0 = [ebuf[b, pl.ds((l0 + j) * EMB + h * 16, 16)]
              for j in range(RS0) for h in range(2)]

        def tbody(gi, w):
            for dt in range(4):
                t = gi * 4 + dt
                for h in range(2):
                    p = [w[2 * j + h] * gbuf[g, t, j, pl.ds(h * 16, 16)]
                         for j in range(RS0)]
                    m3 = jnp.maximum(p[2], jnp.maximum(p[3], p[4]))
                    m5 = jnp.maximum(m3, jnp.maximum(p[1], p[5]))
                    m7 = jnp.maximum(m5, jnp.maximum(p[0], p[6]))
                    obuf[o, pl.ds(t * OUT_C + h * 16, 16)] = m7
                    obuf[o, pl.ds(t * OUT_C + EMB + h * 16, 16)] = m5
                    obuf[o, pl.ds(t * OUT_C + 2 * EMB + h * 16, 16)] = m3
                nxt = [ebuf[b, pl.ds((l0 + RS0 + t) * EMB + h * 16, 16)]
                       for h in range(2)]
                w = tuple(w[2:]) + tuple(nxt)
            return w

        lax.fori_loop(0, CHUNK // 4, tbody, tuple(w0))

    pltpu.sync_copy(seq_hbm.at[pl.ds(base_row, ROWS_PER_W), :], idx_all)
    emb_desc(base_row + 0, 0, 0).start()
    emb_desc(base_row + 1, 1, 0).start()
    gather_desc(0, 0).start()

    def qbody(q, carry):
        for s in range(2):
            rp = 2 * q + s
            row0 = base_row + 2 * rp
            if s == 0:
                emb_desc(row0 + 2, 2, 1).start()
                emb_desc(row0 + 3, 3, 1).start()
            else:
                @pl.when(q < 7)
                def _():
                    emb_desc(row0 + 2, 0, 0).start()
                    emb_desc(row0 + 3, 1, 0).start()
            emb_desc(row0, 2 * s, s).wait()
            emb_desc(row0 + 1, 2 * s + 1, s).wait()
            for k in range(10):
                ci = rp * 10 + k
                rr = k // 5
                l0 = (k % 5) * CHUNK
                par = k % 2
                npar = (k + 1) % 2
                if s == 1 and k == 9:
                    @pl.when(q < 7)
                    def _():
                        gather_desc(ci + 1, npar).start()
                else:
                    gather_desc(ci + 1, npar).start()
                gather_desc(ci, par).wait()
                if k < 2 and s == 0:
                    @pl.when(q > 0)
                    def _():
                        out_desc(ci - 2, par).wait()
                else:
                    out_desc(ci - 2, par).wait()
                compute_chunk(2 * s + rr, l0, par, par)
                out_desc(ci, par).start()
        return carry

    lax.fori_loop(0, 8, qbody, 0)
    out_desc(NCH_W - 2, 0).wait()
    out_desc(NCH_W - 1, 1).wait()


_sc_kernel = functools.partial(
    pl.kernel,
    mesh=plsc.VectorSubcoreMesh(core_axis_name="c", subcore_axis_name="s"),
    compiler_params=pltpu.CompilerParams(use_tc_tiling_on_sc=False),
    out_type=jax.ShapeDtypeStruct((BATCH * SEQ * OUT_C,), jnp.float32),
    scratch_types=[
        pltpu.VMEM((ROWS_PER_W, SEQ), jnp.int32),
        pltpu.VMEM((4, EBUF_T * EMB), jnp.float32),
        pltpu.VMEM((2, CHUNK, RS0, EMB), jnp.float32),
        pltpu.VMEM((2, CHUNK * OUT_C), jnp.float32),
        pltpu.SemaphoreType.DMA,
        pltpu.SemaphoreType.DMA,
        pltpu.SemaphoreType.DMA,
        pltpu.SemaphoreType.DMA,
        pltpu.SemaphoreType.DMA,
        pltpu.SemaphoreType.DMA,
    ],
)(_sc_body)


def kernel(seq, seq_emb, U):
    out = _sc_kernel(seq, seq_emb.reshape(BATCH * SEQ * EMB), U)
    return out.reshape(BATCH, SEQ, OUT_C)


# speedup vs baseline: 1.0559x; 1.0559x over previous
import functools

import jax
import jax.numpy as jnp
from jax import lax
from jax.experimental import pallas as pl
from jax.experimental.pallas import tpu as pltpu
from jax.experimental.pallas import tpu_sc as plsc

VOCAB = 100000
EMB = 32
RS0 = 7
RADIUS = RS0 // 2
BATCH = 1024
SEQ = 200
OUT_C = 96

NW = 32
ROWS_PER_W = BATCH // NW
CHUNK = 40
NCHUNK = SEQ // CHUNK
NCH_W = ROWS_PER_W * NCHUNK
EBUF_T = SEQ + 2 * RADIUS + 2


def _sc_body(seq_hbm, emb_hbm, u_hbm, out_hbm, idx_all, ebuf, gbuf, obuf,
             gsem0, gsem1, gsem2, gsem3, esem0, esem1, osem0, osem1):
    c_id = lax.axis_index("c")
    s_id = lax.axis_index("s")
    wid = s_id * 2 + c_id
    base_row = wid * ROWS_PER_W
    gsem = (gsem0, gsem1, gsem2, gsem3)
    osem = (osem0, osem1)
    esem = (esem0, esem1)

    zero = jnp.zeros((16,), jnp.float32)
    for b in range(4):
        for i in range(RADIUS * EMB // 16):
            ebuf[b, pl.ds(i * 16, 16)] = zero
            ebuf[b, pl.ds((RADIUS + SEQ) * EMB + i * 16, 16)] = zero

    def gather_desc(ci, par):
        return pltpu.make_async_copy(
            u_hbm.at[idx_all.at[ci // NCHUNK, pl.ds((ci % NCHUNK) * CHUNK, CHUNK)]],
            gbuf.at[par], gsem[par])

    def emb_desc(row, b, par):
        return pltpu.make_async_copy(
            emb_hbm.at[pl.ds(row * (SEQ * EMB), SEQ * EMB)],
            ebuf.at[b, pl.ds(RADIUS * EMB, SEQ * EMB)], esem[par])

    out0 = base_row * SEQ * OUT_C

    def out_desc(ci, par):
        return pltpu.make_async_copy(
            obuf.at[par],
            out_hbm.at[pl.ds(out0 + ci * (CHUNK * OUT_C), CHUNK * OUT_C)],
            osem[par])

    def compute_chunk(b, l0, g, o):
        w0 = [ebuf[b, pl.ds((l0 + j) * EMB + h * 16, 16)]
              for j in range(RS0) for h in range(2)]

        def tbody(t, w):
            for h in range(2):
                p = [w[2 * j + h] * gbuf[g, t, j, pl.ds(h * 16, 16)]
                     for j in range(RS0)]
                m3 = jnp.maximum(p[2], jnp.maximum(p[3], p[4]))
                m5 = jnp.maximum(m3, jnp.maximum(p[1], p[5]))
                m7 = jnp.maximum(m5, jnp.maximum(p[0], p[6]))
                obuf[o, pl.ds(t * OUT_C + h * 16, 16)] = m7
                obuf[o, pl.ds(t * OUT_C + EMB + h * 16, 16)] = m5
                obuf[o, pl.ds(t * OUT_C + 2 * EMB + h * 16, 16)] = m3
            nxt = [ebuf[b, pl.ds((l0 + RS0 + t) * EMB + h * 16, 16)]
                   for h in range(2)]
            return tuple(w[2:]) + tuple(nxt)

        lax.fori_loop(0, CHUNK, tbody, tuple(w0))

    pltpu.sync_copy(seq_hbm.at[pl.ds(base_row, ROWS_PER_W), :], idx_all)
    emb_desc(base_row + 0, 0, 0).start()
    emb_desc(base_row + 1, 1, 0).start()
    gather_desc(0, 0).start()
    gather_desc(1, 1).start()
    gather_desc(2, 2).start()

    def qbody(q, carry):
        for s in range(2):
            rp = 2 * q + s
            row0 = base_row + 2 * rp
            if s == 0:
                emb_desc(row0 + 2, 2, 1).start()
                emb_desc(row0 + 3, 3, 1).start()
            else:
                @pl.when(q < 7)
                def _():
                    emb_desc(row0 + 2, 0, 0).start()
                    emb_desc(row0 + 3, 1, 0).start()
            emb_desc(row0, 2 * s, s).wait()
            emb_desc(row0 + 1, 2 * s + 1, s).wait()
            for k in range(10):
                ci = rp * 10 + k
                rr = k // 5
                l0 = (k % 5) * CHUNK
                par = k % 2
                gpar = (10 * s + k) % 4
                gpar3 = (10 * s + k + 3) % 4
                if s == 1 and k >= 7:
                    @pl.when(q < 7)
                    def _():
                        gather_desc(ci + 3, gpar3).start()
                else:
                    gather_desc(ci + 3, gpar3).start()
                gather_desc(ci, gpar).wait()
                if k < 2 and s == 0:
                    @pl.when(q > 0)
                    def _():
                        out_desc(ci - 2, par).wait()
                else:
                    out_desc(ci - 2, par).wait()
                compute_chunk(2 * s + rr, l0, gpar, par)
                out_desc(ci, par).start()
        return carry

    lax.fori_loop(0, 8, qbody, 0)
    out_desc(NCH_W - 2, 0).wait()
    out_desc(NCH_W - 1, 1).wait()


_sc_kernel = functools.partial(
    pl.kernel,
    mesh=plsc.VectorSubcoreMesh(core_axis_name="c", subcore_axis_name="s"),
    compiler_params=pltpu.CompilerParams(use_tc_tiling_on_sc=False),
    out_type=jax.ShapeDtypeStruct((BATCH * SEQ * OUT_C,), jnp.float32),
    scratch_types=[
        pltpu.VMEM((ROWS_PER_W, SEQ), jnp.int32),
        pltpu.VMEM((4, EBUF_T * EMB), jnp.float32),
        pltpu.VMEM((4, CHUNK, RS0, EMB), jnp.float32),
        pltpu.VMEM((2, CHUNK * OUT_C), jnp.float32),
        pltpu.SemaphoreType.DMA,
        pltpu.SemaphoreType.DMA,
        pltpu.SemaphoreType.DMA,
        pltpu.SemaphoreType.DMA,
        pltpu.SemaphoreType.DMA,
        pltpu.SemaphoreType.DMA,
        pltpu.SemaphoreType.DMA,
        pltpu.SemaphoreType.DMA,
    ],
)(_sc_body)


def kernel(seq, seq_emb, U):
    out = _sc_kernel(seq, seq_emb.reshape(BATCH * SEQ * EMB), U)
    return out.reshape(BATCH, SEQ, OUT_C)
